# baseline (speedup 1.0000x reference)
import functools

import jax
import jax.numpy as jnp
from jax import lax
from jax.experimental import pallas as pl
from jax.experimental.pallas import tpu as pltpu
from jax.experimental.pallas import tpu_sc as plsc

N = 10000
D = 256
E = 160000

ROWS_MM = 1000
ROWS_SM = 16

NC = 2
NS = 16
NW = NC * NS
EPW = E // NW
CH = 40
NCHUNK = EPW // CH

RB = 80
NBLK = 125
NBLK_IT = 63
BLK_W = RB * N
SP_SZ = BLK_W + 128
DUMP = BLK_W
EPT = E // NS
EPT_PAD = 10240
NROW128 = EPT_PAD // 128
ZSPAN = SP_SZ // NS
Z_FULL = ZSPAN // 5008
Z_REM = ZSPAN - Z_FULL * 5008
CSPAN = BLK_W // NS
BNC = 5000

NEG = -9e15


def _matmul_body(x_ref, w_ref, o_ref, rn_ref):
    xb = jnp.dot(x_ref[...], w_ref[...], preferred_element_type=jnp.float32)
    o_ref[...] = xb
    rn_ref[...] = 1.0 / jnp.sqrt(jnp.sum(xb * xb, axis=1, keepdims=True))


def _project(x, W):
    return pl.pallas_call(
        _matmul_body,
        grid=(N // ROWS_MM,),
        in_specs=[
            pl.BlockSpec((ROWS_MM, D), lambda i: (i, 0)),
            pl.BlockSpec((D, D), lambda i: (0, 0)),
        ],
        out_specs=[
            pl.BlockSpec((ROWS_MM, D), lambda i: (i, 0)),
            pl.BlockSpec((ROWS_MM, 1), lambda i: (i, 0)),
        ],
        out_shape=[
            jax.ShapeDtypeStruct((N, D), jnp.float32),
            jax.ShapeDtypeStruct((N, 1), jnp.float32),
        ],
    )(x, W)


def _score_body(x_hbm, rn_hbm, src_hbm, dst_hbm, a_hbm, s_hbm,
                sidx, didx, xs_b, xd_b, rs_b, rd_b, s_buf, sm, a_buf,
                sem_s, sem_d, sem_rs, sem_rd):
    wid = lax.axis_index("s") * NC + lax.axis_index("c")
    base = wid * EPW
    pltpu.sync_copy(src_hbm.at[pl.ds(base, EPW)], sidx)
    pltpu.sync_copy(dst_hbm.at[pl.ds(base, EPW)], didx)
    pltpu.sync_copy(a_hbm, a_buf)
    iota = lax.iota(jnp.int32, 16)
    zero_v = jnp.zeros((16,), jnp.float32)

    def bf16_round(v):
        u = plsc.bitcast(v, jnp.int32)
        u = (u + 0x7FFF + ((u >> 16) & 1)) & (-65536)
        return plsc.bitcast(u, jnp.float32)

    ab_v = [bf16_round(a_buf[pl.ds(k * 16, 16)]) for k in range(16)]

    def chunk(c, _):
        off = c * CH
        cp1 = pltpu.async_copy(x_hbm.at[sidx.at[pl.ds(off, CH)]], xs_b, sem_s)
        cp2 = pltpu.async_copy(x_hbm.at[didx.at[pl.ds(off, CH)]], xd_b, sem_d)
        cp3 = pltpu.async_copy(rn_hbm.at[sidx.at[pl.ds(off, CH)]],
                               rs_b.at[pl.ds(0, CH)], sem_rs)
        cp4 = pltpu.async_copy(rn_hbm.at[didx.at[pl.ds(off, CH)]],
                               rd_b.at[pl.ds(0, CH)], sem_rd)
        cp1.wait()
        cp2.wait()

        cp3.wait()
        cp4.wait()

        def edge(e, _):
            acc = zero_v
            rv = (rs_b[pl.ds(e, 16)] * rd_b[pl.ds(e, 16)])[0]
            for k in range(16):
                xs = xs_b[e, pl.ds(k * 16, 16)]
                xd = xd_b[e, pl.ds(k * 16, 16)]
                h = jnp.maximum(xs * xd, 0.0) * rv
                acc = acc + bf16_round(h) * ab_v[k]
            sm[pl.ds(e * 16, 16)] = acc
            return 0

        lax.fori_loop(0, CH, edge, 0, unroll=False)

        for g in range(CH // 16 + 1):
            gbase = (g * 16 + iota) * 16
            tot = zero_v
            for k in range(16):
                tot = tot + plsc.load_gather(sm, [gbase + k])
            sel = jnp.where(tot > 0, tot, NEG)
            s_buf[pl.ds(off + g * 16, 16)] = sel
        return 0

    lax.fori_loop(0, NCHUNK, chunk, 0, unroll=False)
    pltpu.sync_copy(s_buf.at[pl.ds(0, EPW)], s_hbm.at[pl.ds(base, EPW)])


def _score_sc(x, rn, src, dst, a_vec):
    mesh = plsc.VectorSubcoreMesh(core_axis_name="c", subcore_axis_name="s")
    f = pl.kernel(
        _score_body,
        out_type=jax.ShapeDtypeStruct((E,), jnp.float32),
        mesh=mesh,
        compiler_params=pltpu.CompilerParams(needs_layout_passes=False),
        scratch_types=[
            pltpu.VMEM((EPW,), jnp.int32),
            pltpu.VMEM((EPW,), jnp.int32),
            pltpu.VMEM((CH, D), jnp.float32),
            pltpu.VMEM((CH, D), jnp.float32),
            pltpu.VMEM((CH + 16,), jnp.float32),
            pltpu.VMEM((CH + 16,), jnp.float32),
            pltpu.VMEM((EPW + 16,), jnp.float32),
            pltpu.VMEM(((CH + 16) * 16,), jnp.float32),
            pltpu.VMEM((D,), jnp.float32),
            pltpu.SemaphoreType.DMA,
            pltpu.SemaphoreType.DMA,
            pltpu.SemaphoreType.DMA,
            pltpu.SemaphoreType.DMA,
        ],
    )
    return f(x, rn, src, dst, a_vec)


def _assemble_body(src_hbm, dst_hbm, s_hbm, m_hbm,
                   key_r, dst_r, s_r, lidx, zbuf, bnc0, bnc1, spmem,
                   bsem0, bsem1):
    core = lax.axis_index("c")
    sub = lax.axis_index("s")
    base = sub * EPT
    pltpu.sync_copy(src_hbm.at[pl.ds(base, EPT)], key_r.at[pl.ds(0, EPT)])
    pltpu.sync_copy(dst_hbm.at[pl.ds(base, EPT)], dst_r.at[pl.ds(0, EPT)])
    pltpu.sync_copy(s_hbm.at[pl.ds(base, EPT)], s_r.at[pl.ds(0, EPT)])

    def keyset(i, _):
        key_r[pl.ds(i * 16, 16)] = key_r[pl.ds(i * 16, 16)] * N + dst_r[pl.ds(i * 16, 16)]
        return 0

    lax.fori_loop(0, EPT // 16, keyset, 0, unroll=False)
    neg1 = jnp.full((16,), -1, jnp.int32)
    for i in range(EPT // 16, EPT_PAD // 16):
        key_r[pl.ds(i * 16, 16)] = neg1

    def zset(i, _):
        zbuf[pl.ds(i * 16, 16)] = jnp.zeros((16,), jnp.float32)
        return 0

    lax.fori_loop(0, 5120 // 16, zset, 0, unroll=False)

    zb = sub * ZSPAN

    def zinit(i, _):
        pltpu.sync_copy(zbuf.at[pl.ds(0, 5008)],
                        spmem.at[pl.ds(zb + i * 5008, 5008)])
        return 0

    lax.fori_loop(0, Z_FULL, zinit, 0, unroll=False)
    pltpu.sync_copy(zbuf.at[pl.ds(0, Z_REM)],
                    spmem.at[pl.ds(zb + Z_FULL * 5008, Z_REM)])
    plsc.subcore_barrier()

    dump_v = jnp.full((16,), DUMP, jnp.int32)

    def block(b, _):
        lo = (b * NC + core) * RB
        lon = lo * N
        hin = jnp.minimum(lon + RB * N, N * N)

        def idxrow(r, _):
            for kk in range(8):
                o = r * 128 + kk * 16
                k = key_r[pl.ds(o, 16)]
                ok = (k >= lon) & (k < hin)
                lidx[r, pl.ds(kk * 16, 16)] = jnp.where(ok, k - lon, dump_v)
            return 0

        lax.fori_loop(0, NROW128, idxrow, 0, unroll=False)

        def sadd(j, _):
            pltpu.sync_copy(s_r.at[pl.ds(j * 128, 128)],
                            spmem.at[lidx.at[j]], add=True)
            return 0

        lax.fori_loop(0, NROW128, sadd, 0, unroll=False)
        plsc.subcore_barrier()

        sbase = sub * CSPAN
        hbase = lon + sub * CSPAN

        @pl.when(lo < N)
        def _copy_out():
            pend = [None, None]
            for i in range(CSPAN // BNC):
                bb = (bnc0, bnc1)[i % 2]
                if pend[i % 2] is not None:
                    pend[i % 2].wait()
                pltpu.sync_copy(spmem.at[pl.ds(sbase + i * BNC, BNC)], bb)
                pend[i % 2] = pltpu.async_copy(
                    bb, m_hbm.at[pl.ds(hbase + i * BNC, BNC)],
                    (bsem0, bsem1)[i % 2])
            pend[0].wait()
            pend[1].wait()

        plsc.subcore_barrier()

        def szero(j, _):
            pltpu.sync_copy(zbuf.at[pl.ds(0, 128)], spmem.at[lidx.at[j]])
            return 0

        lax.fori_loop(0, NROW128, szero, 0, unroll=False)
        plsc.subcore_barrier()
        return 0

    lax.fori_loop(0, NBLK_IT, block, 0, unroll=False)


def _assemble_sc(src, dst, s):
    mesh = plsc.VectorSubcoreMesh(core_axis_name="c", subcore_axis_name="s")
    f = pl.kernel(
        _assemble_body,
        out_type=jax.ShapeDtypeStruct((N * N,), jnp.float32),
        mesh=mesh,
        compiler_params=pltpu.CompilerParams(needs_layout_passes=False),
        scratch_types=[
            pltpu.VMEM((EPT_PAD,), jnp.int32),
            pltpu.VMEM((EPT_PAD,), jnp.int32),
            pltpu.VMEM((EPT_PAD,), jnp.float32),
            pltpu.VMEM((NROW128, 128), jnp.int32),
            pltpu.VMEM((5120,), jnp.float32),
            pltpu.VMEM((BNC,), jnp.float32),
            pltpu.VMEM((BNC,), jnp.float32),
            pltpu.VMEM_SHARED((SP_SZ,), jnp.float32),
            pltpu.SemaphoreType.DMA,
            pltpu.SemaphoreType.DMA,
        ],
    )
    return f(src, dst, s)


def _softmax_body(m_ref, o_ref):
    m = m_ref[...]
    mask = m != 0.0
    logits = jnp.where(mask, m, -jnp.inf)
    rowmax = jnp.max(logits, axis=1, keepdims=True)
    safe = jnp.where(jnp.isfinite(rowmax), rowmax, 0.0)
    e = jnp.where(mask, jnp.exp(m - safe), 0.0)
    denom = jnp.sum(e, axis=1, keepdims=True)
    o_ref[...] = jnp.where(denom > 0, e / jnp.where(denom > 0, denom, 1.0), 0.0)


def _row_softmax(M):
    return pl.pallas_call(
        _softmax_body,
        grid=(N // ROWS_SM,),
        in_specs=[pl.BlockSpec((ROWS_SM, N), lambda i: (i, 0))],
        out_specs=pl.BlockSpec((ROWS_SM, N), lambda i: (i, 0)),
        out_shape=jax.ShapeDtypeStruct((N, N), jnp.float32),
    )(M)


def _score_jnp(x, src, dst, a):
    xs = x[src]
    xd = x[dst]
    norm = jnp.sqrt(
        jnp.sum(xs * xs, axis=1, keepdims=True)
        * jnp.sum(xd * xd, axis=1, keepdims=True))
    h = jax.nn.relu(xs * xd / norm)
    s = jnp.squeeze(h @ a)
    return jnp.where(s > 0, s, jnp.full_like(s, NEG))


def kernel(input, edge, W, a):
    x, rn = _project(input, W)
    src = edge[0]
    dst = edge[1]
    s = _score_sc(x, rn.reshape(-1), src, dst, a.reshape(-1))
    M = _assemble_sc(src, dst, s)
    A = _row_softmax(M.reshape(N, N))
    return (x, A)


# speedup vs baseline: 1.0003x; 1.0003x over previous
import functools

import jax
import jax.numpy as jnp
from jax import lax
from jax.experimental import pallas as pl
from jax.experimental.pallas import tpu as pltpu
from jax.experimental.pallas import tpu_sc as plsc

N = 10000
D = 256
E = 160000

ROWS_MM = 1000
ROWS_SM = 16

NC = 2
NS = 16
NW = NC * NS
EPW = E // NW
CH = 40
NCHUNK = EPW // CH

RB = 80
NBLK = 125
NBLK_IT = 63
BLK_W = RB * N
SP_SZ = BLK_W + 128
DUMP = BLK_W
EPT = E // NS
EPT_PAD = 10240
NROW128 = EPT_PAD // 128
ZSPAN = SP_SZ // NS
Z_FULL = ZSPAN // 5008
Z_REM = ZSPAN - Z_FULL * 5008
CSPAN = BLK_W // NS
BNC = 5000

NEG = -9e15


def _matmul_body(x_ref, w_ref, o_ref, rn_ref):
    xb = jnp.dot(x_ref[...], w_ref[...], preferred_element_type=jnp.float32)
    o_ref[...] = xb
    rn_ref[...] = 1.0 / jnp.sqrt(jnp.sum(xb * xb, axis=1, keepdims=True))


def _project(x, W):
    return pl.pallas_call(
        _matmul_body,
        grid=(N // ROWS_MM,),
        in_specs=[
            pl.BlockSpec((ROWS_MM, D), lambda i: (i, 0)),
            pl.BlockSpec((D, D), lambda i: (0, 0)),
        ],
        out_specs=[
            pl.BlockSpec((ROWS_MM, D), lambda i: (i, 0)),
            pl.BlockSpec((ROWS_MM, 1), lambda i: (i, 0)),
        ],
        out_shape=[
            jax.ShapeDtypeStruct((N, D), jnp.float32),
            jax.ShapeDtypeStruct((N, 1), jnp.float32),
        ],
    )(x, W)


def _score_body(x_hbm, rn_hbm, src_hbm, dst_hbm, a_hbm, s_hbm,
                sidx, didx, xs_b, xd_b, rs_b, rd_b, s_buf, sm, a_buf,
                sem_s, sem_d, sem_rs, sem_rd):
    wid = lax.axis_index("s") * NC + lax.axis_index("c")
    base = wid * EPW
    pltpu.sync_copy(src_hbm.at[pl.ds(base, EPW)], sidx)
    pltpu.sync_copy(dst_hbm.at[pl.ds(base, EPW)], didx)
    pltpu.sync_copy(a_hbm, a_buf)
    iota = lax.iota(jnp.int32, 16)
    zero_v = jnp.zeros((16,), jnp.float32)

    def bf16_round(v):
        u = plsc.bitcast(v, jnp.int32)
        u = (u + 0x7FFF + ((u >> 16) & 1)) & (-65536)
        return plsc.bitcast(u, jnp.float32)

    ab_v = [bf16_round(a_buf[pl.ds(k * 16, 16)]) for k in range(16)]

    def chunk(c, _):
        off = c * CH
        cp1 = pltpu.async_copy(x_hbm.at[sidx.at[pl.ds(off, CH)]], xs_b, sem_s)
        cp2 = pltpu.async_copy(x_hbm.at[didx.at[pl.ds(off, CH)]], xd_b, sem_d)
        cp3 = pltpu.async_copy(rn_hbm.at[sidx.at[pl.ds(off, CH)]],
                               rs_b.at[pl.ds(0, CH)], sem_rs)
        cp4 = pltpu.async_copy(rn_hbm.at[didx.at[pl.ds(off, CH)]],
                               rd_b.at[pl.ds(0, CH)], sem_rd)
        cp1.wait()
        cp2.wait()

        cp3.wait()
        cp4.wait()

        def edge(e, _):
            acc = zero_v
            rv = (rs_b[pl.ds(e, 16)] * rd_b[pl.ds(e, 16)])[0]
            for k in range(16):
                xs = xs_b[e, pl.ds(k * 16, 16)]
                xd = xd_b[e, pl.ds(k * 16, 16)]
                h = jnp.maximum(xs * xd, 0.0) * rv
                acc = acc + bf16_round(h) * ab_v[k]
            sm[pl.ds(e * 16, 16)] = acc
            return 0

        lax.fori_loop(0, CH, edge, 0, unroll=False)

        for g in range(CH // 16 + 1):
            gbase = (g * 16 + iota) * 16
            tot = zero_v
            for k in range(16):
                tot = tot + plsc.load_gather(sm, [gbase + k])
            sel = jnp.where(tot > 0, tot, NEG)
            s_buf[pl.ds(off + g * 16, 16)] = sel
        return 0

    lax.fori_loop(0, NCHUNK, chunk, 0, unroll=False)
    pltpu.sync_copy(s_buf.at[pl.ds(0, EPW)], s_hbm.at[pl.ds(base, EPW)])


def _score_sc(x, rn, src, dst, a_vec):
    mesh = plsc.VectorSubcoreMesh(core_axis_name="c", subcore_axis_name="s")
    f = pl.kernel(
        _score_body,
        out_type=jax.ShapeDtypeStruct((E,), jnp.float32),
        mesh=mesh,
        compiler_params=pltpu.CompilerParams(needs_layout_passes=False),
        scratch_types=[
            pltpu.VMEM((EPW,), jnp.int32),
            pltpu.VMEM((EPW,), jnp.int32),
            pltpu.VMEM((CH, D), jnp.float32),
            pltpu.VMEM((CH, D), jnp.float32),
            pltpu.VMEM((CH + 16,), jnp.float32),
            pltpu.VMEM((CH + 16,), jnp.float32),
            pltpu.VMEM((EPW + 16,), jnp.float32),
            pltpu.VMEM(((CH + 16) * 16,), jnp.float32),
            pltpu.VMEM((D,), jnp.float32),
            pltpu.SemaphoreType.DMA,
            pltpu.SemaphoreType.DMA,
            pltpu.SemaphoreType.DMA,
            pltpu.SemaphoreType.DMA,
        ],
    )
    return f(x, rn, src, dst, a_vec)


def _assemble_body(src_hbm, dst_hbm, s_hbm, m_hbm,
                   key_r, dst_r, s_r, lidx, zbuf, bnc0, bnc1, spmem,
                   bsem0, bsem1):
    core = lax.axis_index("c")
    sub = lax.axis_index("s")
    base = sub * EPT
    pltpu.sync_copy(src_hbm.at[pl.ds(base, EPT)], key_r.at[pl.ds(0, EPT)])
    pltpu.sync_copy(dst_hbm.at[pl.ds(base, EPT)], dst_r.at[pl.ds(0, EPT)])
    pltpu.sync_copy(s_hbm.at[pl.ds(base, EPT)], s_r.at[pl.ds(0, EPT)])

    def keyset(i, _):
        key_r[pl.ds(i * 16, 16)] = key_r[pl.ds(i * 16, 16)] * N + dst_r[pl.ds(i * 16, 16)]
        return 0

    lax.fori_loop(0, EPT // 16, keyset, 0, unroll=False)
    neg1 = jnp.full((16,), -1, jnp.int32)
    for i in range(EPT // 16, EPT_PAD // 16):
        key_r[pl.ds(i * 16, 16)] = neg1

    def zset(i, _):
        zbuf[pl.ds(i * 16, 16)] = jnp.zeros((16,), jnp.float32)
        return 0

    lax.fori_loop(0, 5120 // 16, zset, 0, unroll=False)

    zb = sub * ZSPAN

    def zinit(i, _):
        pltpu.sync_copy(zbuf.at[pl.ds(0, 5008)],
                        spmem.at[pl.ds(zb + i * 5008, 5008)])
        return 0

    lax.fori_loop(0, Z_FULL, zinit, 0, unroll=False)
    pltpu.sync_copy(zbuf.at[pl.ds(0, Z_REM)],
                    spmem.at[pl.ds(zb + Z_FULL * 5008, Z_REM)])
    plsc.subcore_barrier()

    dump_v = jnp.full((16,), DUMP, jnp.int32)

    def block(b, _):
        lo = (b * NC + core) * RB
        lon = lo * N
        hin = jnp.minimum(lon + RB * N, N * N)

        def idxrow(r, _):
            for kk in range(8):
                o = r * 128 + kk * 16
                k = key_r[pl.ds(o, 16)]
                ok = (k >= lon) & (k < hin)
                lidx[r, pl.ds(kk * 16, 16)] = jnp.where(ok, k - lon, dump_v)
            return 0

        lax.fori_loop(0, NROW128, idxrow, 0, unroll=False)

        def sadd(j, _):
            pltpu.async_copy(s_r.at[pl.ds(j * 128, 128)],
                             spmem.at[lidx.at[j]], bsem0, add=True)
            return 0

        lax.fori_loop(0, NROW128, sadd, 0, unroll=False)

        def sadd_drain(j, _):
            pltpu.make_async_copy(s_r.at[pl.ds(j * 128, 128)],
                                  spmem.at[lidx.at[j]], bsem0).wait()
            return 0

        lax.fori_loop(0, NROW128, sadd_drain, 0, unroll=False)
        plsc.subcore_barrier()

        sbase = sub * CSPAN
        hbase = lon + sub * CSPAN

        @pl.when(lo < N)
        def _copy_out():
            pend = [None, None]
            for i in range(CSPAN // BNC):
                bb = (bnc0, bnc1)[i % 2]
                if pend[i % 2] is not None:
                    pend[i % 2].wait()
                pltpu.sync_copy(spmem.at[pl.ds(sbase + i * BNC, BNC)], bb)
                pend[i % 2] = pltpu.async_copy(
                    bb, m_hbm.at[pl.ds(hbase + i * BNC, BNC)],
                    (bsem0, bsem1)[i % 2])
            pend[0].wait()
            pend[1].wait()

        plsc.subcore_barrier()

        def szero(j, _):
            pltpu.async_copy(zbuf.at[pl.ds(0, 128)], spmem.at[lidx.at[j]],
                             bsem1)
            return 0

        lax.fori_loop(0, NROW128, szero, 0, unroll=False)

        def szero_drain(j, _):
            pltpu.make_async_copy(zbuf.at[pl.ds(0, 128)],
                                  spmem.at[lidx.at[j]], bsem1).wait()
            return 0

        lax.fori_loop(0, NROW128, szero_drain, 0, unroll=False)
        plsc.subcore_barrier()
        return 0

    lax.fori_loop(0, NBLK_IT, block, 0, unroll=False)


def _assemble_sc(src, dst, s):
    mesh = plsc.VectorSubcoreMesh(core_axis_name="c", subcore_axis_name="s")
    f = pl.kernel(
        _assemble_body,
        out_type=jax.ShapeDtypeStruct((N * N,), jnp.float32),
        mesh=mesh,
        compiler_params=pltpu.CompilerParams(needs_layout_passes=False),
        scratch_types=[
            pltpu.VMEM((EPT_PAD,), jnp.int32),
            pltpu.VMEM((EPT_PAD,), jnp.int32),
            pltpu.VMEM((EPT_PAD,), jnp.float32),
            pltpu.VMEM((NROW128, 128), jnp.int32),
            pltpu.VMEM((5120,), jnp.float32),
            pltpu.VMEM((BNC,), jnp.float32),
            pltpu.VMEM((BNC,), jnp.float32),
            pltpu.VMEM_SHARED((SP_SZ,), jnp.float32),
            pltpu.SemaphoreType.DMA,
            pltpu.SemaphoreType.DMA,
        ],
    )
    return f(src, dst, s)


def _softmax_body(m_ref, o_ref):
    m = m_ref[...]
    mask = m != 0.0
    logits = jnp.where(mask, m, -jnp.inf)
    rowmax = jnp.max(logits, axis=1, keepdims=True)
    safe = jnp.where(jnp.isfinite(rowmax), rowmax, 0.0)
    e = jnp.where(mask, jnp.exp(m - safe), 0.0)
    denom = jnp.sum(e, axis=1, keepdims=True)
    o_ref[...] = jnp.where(denom > 0, e / jnp.where(denom > 0, denom, 1.0), 0.0)


def _row_softmax(M):
    return pl.pallas_call(
        _softmax_body,
        grid=(N // ROWS_SM,),
        in_specs=[pl.BlockSpec((ROWS_SM, N), lambda i: (i, 0))],
        out_specs=pl.BlockSpec((ROWS_SM, N), lambda i: (i, 0)),
        out_shape=jax.ShapeDtypeStruct((N, N), jnp.float32),
    )(M)


def _score_jnp(x, src, dst, a):
    xs = x[src]
    xd = x[dst]
    norm = jnp.sqrt(
        jnp.sum(xs * xs, axis=1, keepdims=True)
        * jnp.sum(xd * xd, axis=1, keepdims=True))
    h = jax.nn.relu(xs * xd / norm)
    s = jnp.squeeze(h @ a)
    return jnp.where(s > 0, s, jnp.full_like(s, NEG))


def kernel(input, edge, W, a):
    x, rn = _project(input, W)
    src = edge[0]
    dst = edge[1]
    s = _score_sc(x, rn.reshape(-1), src, dst, a.reshape(-1))
    M = _assemble_sc(src, dst, s)
    A = _row_softmax(M.reshape(N, N))
    return (x, A)


# speedup vs baseline: 9.3621x; 9.3590x over previous
import functools

import jax
import jax.numpy as jnp
from jax import lax
from jax.experimental import pallas as pl
from jax.experimental.pallas import tpu as pltpu
from jax.experimental.pallas import tpu_sc as plsc

N = 10000
D = 256
E = 160000

ROWS_MM = 1000
ROWS_SM = 16

NC = 2
NS = 16
NW = NC * NS
EPW = E // NW
CH = 40
NCHUNK = EPW // CH

RB = 80
NBLK = 125
NBLK_IT = 63
BLK_W = RB * N
SP_SZ = BLK_W + 128
DUMP = BLK_W
EPT = E // NS
EPT_PAD = 10240
NROW128 = EPT_PAD // 128
ZSPAN = SP_SZ // NS
Z_FULL = ZSPAN // 5008
Z_REM = ZSPAN - Z_FULL * 5008
CSPAN = BLK_W // NS
BNC = 5000

NEG = -9e15


def _matmul_body(x_ref, w_ref, o_ref, rn_ref):
    xb = jnp.dot(x_ref[...], w_ref[...], preferred_element_type=jnp.float32)
    o_ref[...] = xb
    rn_ref[...] = 1.0 / jnp.sqrt(jnp.sum(xb * xb, axis=1, keepdims=True))


def _project(x, W):
    return pl.pallas_call(
        _matmul_body,
        grid=(N // ROWS_MM,),
        in_specs=[
            pl.BlockSpec((ROWS_MM, D), lambda i: (i, 0)),
            pl.BlockSpec((D, D), lambda i: (0, 0)),
        ],
        out_specs=[
            pl.BlockSpec((ROWS_MM, D), lambda i: (i, 0)),
            pl.BlockSpec((ROWS_MM, 1), lambda i: (i, 0)),
        ],
        out_shape=[
            jax.ShapeDtypeStruct((N, D), jnp.float32),
            jax.ShapeDtypeStruct((N, 1), jnp.float32),
        ],
    )(x, W)


def _score_body(x_hbm, rn_hbm, src_hbm, dst_hbm, a_hbm, s_hbm,
                sidx, didx, xs_b, xd_b, rs_b, rd_b, s_buf, sm, a_buf,
                sem_s, sem_d, sem_rs, sem_rd):
    wid = lax.axis_index("s") * NC + lax.axis_index("c")
    base = wid * EPW
    pltpu.sync_copy(src_hbm.at[pl.ds(base, EPW)], sidx)
    pltpu.sync_copy(dst_hbm.at[pl.ds(base, EPW)], didx)
    pltpu.sync_copy(a_hbm, a_buf)
    iota = lax.iota(jnp.int32, 16)
    zero_v = jnp.zeros((16,), jnp.float32)

    def bf16_round(v):
        u = plsc.bitcast(v, jnp.int32)
        u = (u + 0x7FFF + ((u >> 16) & 1)) & (-65536)
        return plsc.bitcast(u, jnp.float32)

    ab_v = [bf16_round(a_buf[pl.ds(k * 16, 16)]) for k in range(16)]

    def chunk(c, _):
        off = c * CH
        cp1 = pltpu.async_copy(x_hbm.at[sidx.at[pl.ds(off, CH)]], xs_b, sem_s)
        cp2 = pltpu.async_copy(x_hbm.at[didx.at[pl.ds(off, CH)]], xd_b, sem_d)
        cp3 = pltpu.async_copy(rn_hbm.at[sidx.at[pl.ds(off, CH)]],
                               rs_b.at[pl.ds(0, CH)], sem_rs)
        cp4 = pltpu.async_copy(rn_hbm.at[didx.at[pl.ds(off, CH)]],
                               rd_b.at[pl.ds(0, CH)], sem_rd)
        cp1.wait()
        cp2.wait()

        cp3.wait()
        cp4.wait()

        def edge(e, _):
            acc = zero_v
            rv = (rs_b[pl.ds(e, 16)] * rd_b[pl.ds(e, 16)])[0]
            for k in range(16):
                xs = xs_b[e, pl.ds(k * 16, 16)]
                xd = xd_b[e, pl.ds(k * 16, 16)]
                h = jnp.maximum(xs * xd, 0.0) * rv
                acc = acc + bf16_round(h) * ab_v[k]
            sm[pl.ds(e * 16, 16)] = acc
            return 0

        lax.fori_loop(0, CH, edge, 0, unroll=False)

        for g in range(CH // 16 + 1):
            gbase = (g * 16 + iota) * 16
            tot = zero_v
            for k in range(16):
                tot = tot + plsc.load_gather(sm, [gbase + k])
            sel = jnp.where(tot > 0, tot, NEG)
            s_buf[pl.ds(off + g * 16, 16)] = sel
        return 0

    lax.fori_loop(0, NCHUNK, chunk, 0, unroll=False)
    pltpu.sync_copy(s_buf.at[pl.ds(0, EPW)], s_hbm.at[pl.ds(base, EPW)])


def _score_sc(x, rn, src, dst, a_vec):
    mesh = plsc.VectorSubcoreMesh(core_axis_name="c", subcore_axis_name="s")
    f = pl.kernel(
        _score_body,
        out_type=jax.ShapeDtypeStruct((E,), jnp.float32),
        mesh=mesh,
        compiler_params=pltpu.CompilerParams(needs_layout_passes=False),
        scratch_types=[
            pltpu.VMEM((EPW,), jnp.int32),
            pltpu.VMEM((EPW,), jnp.int32),
            pltpu.VMEM((CH, D), jnp.float32),
            pltpu.VMEM((CH, D), jnp.float32),
            pltpu.VMEM((CH + 16,), jnp.float32),
            pltpu.VMEM((CH + 16,), jnp.float32),
            pltpu.VMEM((EPW + 16,), jnp.float32),
            pltpu.VMEM(((CH + 16) * 16,), jnp.float32),
            pltpu.VMEM((D,), jnp.float32),
            pltpu.SemaphoreType.DMA,
            pltpu.SemaphoreType.DMA,
            pltpu.SemaphoreType.DMA,
            pltpu.SemaphoreType.DMA,
        ],
    )
    return f(x, rn, src, dst, a_vec)


def _assemble_body(src_hbm, dst_hbm, s_hbm, m_hbm,
                   key_r, dst_r, s_r, lidx, zbuf, bnc0, bnc1, spmem,
                   bsem0, bsem1):
    core = lax.axis_index("c")
    sub = lax.axis_index("s")
    base = sub * EPT
    pltpu.sync_copy(src_hbm.at[pl.ds(base, EPT)], key_r.at[pl.ds(0, EPT)])
    pltpu.sync_copy(dst_hbm.at[pl.ds(base, EPT)], dst_r.at[pl.ds(0, EPT)])
    pltpu.sync_copy(s_hbm.at[pl.ds(base, EPT)], s_r.at[pl.ds(0, EPT)])

    def keyset(i, _):
        key_r[pl.ds(i * 16, 16)] = key_r[pl.ds(i * 16, 16)] * N + dst_r[pl.ds(i * 16, 16)]
        return 0

    lax.fori_loop(0, EPT // 16, keyset, 0, unroll=False)
    neg1 = jnp.full((16,), -1, jnp.int32)
    for i in range(EPT // 16, EPT_PAD // 16):
        key_r[pl.ds(i * 16, 16)] = neg1

    def zset(i, _):
        zbuf[pl.ds(i * 16, 16)] = jnp.zeros((16,), jnp.float32)
        return 0

    lax.fori_loop(0, 5120 // 16, zset, 0, unroll=False)

    zb = sub * ZSPAN

    def zinit(i, _):
        pltpu.sync_copy(zbuf.at[pl.ds(0, 5008)],
                        spmem.at[pl.ds(zb + i * 5008, 5008)])
        return 0

    lax.fori_loop(0, Z_FULL, zinit, 0, unroll=False)
    pltpu.sync_copy(zbuf.at[pl.ds(0, Z_REM)],
                    spmem.at[pl.ds(zb + Z_FULL * 5008, Z_REM)])
    plsc.subcore_barrier()

    iota16 = lax.iota(jnp.int32, 16)

    def block(b, _):
        lo = (b * NC + core) * RB
        lon = lo * N
        hin = jnp.minimum(lon + RB * N, N * N)

        def idxrow(r, _):
            for kk in range(8):
                o = r * 128 + kk * 16
                k = key_r[pl.ds(o, 16)]
                ok = (k >= lon) & (k < hin)
                dmp = DUMP + iota16 + kk * 16
                lidx[r, pl.ds(kk * 16, 16)] = jnp.where(ok, k - lon, dmp)
            return 0

        lax.fori_loop(0, NROW128, idxrow, 0, unroll=False)

        def sadd(j, _):
            pltpu.async_copy(s_r.at[pl.ds(j * 128, 128)],
                             spmem.at[lidx.at[j]], bsem0, add=True)
            return 0

        lax.fori_loop(0, NROW128, sadd, 0, unroll=False)

        def sadd_drain(j, _):
            pltpu.make_async_copy(s_r.at[pl.ds(j * 128, 128)],
                                  spmem.at[lidx.at[j]], bsem0).wait()
            return 0

        lax.fori_loop(0, NROW128, sadd_drain, 0, unroll=False)
        plsc.subcore_barrier()

        sbase = sub * CSPAN
        hbase = lon + sub * CSPAN

        @pl.when(lo < N)
        def _copy_out():
            pend = [None, None]
            for i in range(CSPAN // BNC):
                bb = (bnc0, bnc1)[i % 2]
                if pend[i % 2] is not None:
                    pend[i % 2].wait()
                pltpu.sync_copy(spmem.at[pl.ds(sbase + i * BNC, BNC)], bb)
                pend[i % 2] = pltpu.async_copy(
                    bb, m_hbm.at[pl.ds(hbase + i * BNC, BNC)],
                    (bsem0, bsem1)[i % 2])
            pend[0].wait()
            pend[1].wait()

        plsc.subcore_barrier()

        def szero(j, _):
            pltpu.async_copy(zbuf.at[pl.ds(0, 128)], spmem.at[lidx.at[j]],
                             bsem1)
            return 0

        lax.fori_loop(0, NROW128, szero, 0, unroll=False)

        def szero_drain(j, _):
            pltpu.make_async_copy(zbuf.at[pl.ds(0, 128)],
                                  spmem.at[lidx.at[j]], bsem1).wait()
            return 0

        lax.fori_loop(0, NROW128, szero_drain, 0, unroll=False)
        plsc.subcore_barrier()
        return 0

    lax.fori_loop(0, NBLK_IT, block, 0, unroll=False)


def _assemble_sc(src, dst, s):
    mesh = plsc.VectorSubcoreMesh(core_axis_name="c", subcore_axis_name="s")
    f = pl.kernel(
        _assemble_body,
        out_type=jax.ShapeDtypeStruct((N * N,), jnp.float32),
        mesh=mesh,
        compiler_params=pltpu.CompilerParams(needs_layout_passes=False),
        scratch_types=[
            pltpu.VMEM((EPT_PAD,), jnp.int32),
            pltpu.VMEM((EPT_PAD,), jnp.int32),
            pltpu.VMEM((EPT_PAD,), jnp.float32),
            pltpu.VMEM((NROW128, 128), jnp.int32),
            pltpu.VMEM((5120,), jnp.float32),
            pltpu.VMEM((BNC,), jnp.float32),
            pltpu.VMEM((BNC,), jnp.float32),
            pltpu.VMEM_SHARED((SP_SZ,), jnp.float32),
            pltpu.SemaphoreType.DMA,
            pltpu.SemaphoreType.DMA,
        ],
    )
    return f(src, dst, s)


def _softmax_body(m_ref, o_ref):
    m = m_ref[...]
    mask = m != 0.0
    logits = jnp.where(mask, m, -jnp.inf)
    rowmax = jnp.max(logits, axis=1, keepdims=True)
    safe = jnp.where(jnp.isfinite(rowmax), rowmax, 0.0)
    e = jnp.where(mask, jnp.exp(m - safe), 0.0)
    denom = jnp.sum(e, axis=1, keepdims=True)
    o_ref[...] = jnp.where(denom > 0, e / jnp.where(denom > 0, denom, 1.0), 0.0)


def _row_softmax(M):
    return pl.pallas_call(
        _softmax_body,
        grid=(N // ROWS_SM,),
        in_specs=[pl.BlockSpec((ROWS_SM, N), lambda i: (i, 0))],
        out_specs=pl.BlockSpec((ROWS_SM, N), lambda i: (i, 0)),
        out_shape=jax.ShapeDtypeStruct((N, N), jnp.float32),
    )(M)


def _score_jnp(x, src, dst, a):
    xs = x[src]
    xd = x[dst]
    norm = jnp.sqrt(
        jnp.sum(xs * xs, axis=1, keepdims=True)
        * jnp.sum(xd * xd, axis=1, keepdims=True))
    h = jax.nn.relu(xs * xd / norm)
    s = jnp.squeeze(h @ a)
    return jnp.where(s > 0, s, jnp.full_like(s, NEG))


def kernel(input, edge, W, a):
    x, rn = _project(input, W)
    src = edge[0]
    dst = edge[1]
    s = _score_sc(x, rn.reshape(-1), src, dst, a.reshape(-1))
    M = _assemble_sc(src, dst, s)
    A = _row_softmax(M.reshape(N, N))
    return (x, A)


# speedup vs baseline: 9.3627x; 1.0001x over previous
import jax
import jax.numpy as jnp
from jax import lax
from jax.experimental import pallas as pl
from jax.experimental.pallas import tpu as pltpu
from jax.experimental.pallas import tpu_sc as plsc

N = 10000
D = 256
E = 160000

ROWS_MM = 1000
ROWS_SM = 16

NC = 2
NS = 16
NW = NC * NS
EPW = E // NW
CH = 40
NCHUNK = EPW // CH

RB = 80
NBLK = 125
NBLK_IT = 63
BLK_W = RB * N
SP_SZ = BLK_W + 128
DUMP = BLK_W
EPT = E // NS
EPT_PAD = 10240
NROW128 = EPT_PAD // 128
ZSPAN = SP_SZ // NS
Z_FULL = ZSPAN // 5008
Z_REM = ZSPAN - Z_FULL * 5008
CSPAN = BLK_W // NS
BNC = 5000

NEG = -9e15


def _matmul_body(x_ref, w_ref, o_ref, rn_ref):
    xb = jnp.dot(x_ref[...], w_ref[...], preferred_element_type=jnp.float32)
    o_ref[...] = xb
    rn_ref[...] = 1.0 / jnp.sqrt(jnp.sum(xb * xb, axis=1, keepdims=True))


def _project(x, W):
    return pl.pallas_call(
        _matmul_body,
        grid=(N // ROWS_MM,),
        in_specs=[
            pl.BlockSpec((ROWS_MM, D), lambda i: (i, 0)),
            pl.BlockSpec((D, D), lambda i: (0, 0)),
        ],
        out_specs=[
            pl.BlockSpec((ROWS_MM, D), lambda i: (i, 0)),
            pl.BlockSpec((ROWS_MM, 1), lambda i: (i, 0)),
        ],
        out_shape=[
            jax.ShapeDtypeStruct((N, D), jnp.float32),
            jax.ShapeDtypeStruct((N, 1), jnp.float32),
        ],
    )(x, W)


def _score_body(x_hbm, rn_hbm, src_hbm, dst_hbm, a_hbm, s_hbm,
                sidx, didx, xs_b, xd_b, rs_b, rd_b, s_buf, sm, a_buf,
                sem_s, sem_d, sem_rs, sem_rd):
    wid = lax.axis_index("s") * NC + lax.axis_index("c")
    base = wid * EPW
    pltpu.sync_copy(src_hbm.at[pl.ds(base, EPW)], sidx)
    pltpu.sync_copy(dst_hbm.at[pl.ds(base, EPW)], didx)
    pltpu.sync_copy(a_hbm, a_buf)
    iota = lax.iota(jnp.int32, 16)
    zero_v = jnp.zeros((16,), jnp.float32)

    def bf16_round(v):
        u = plsc.bitcast(v, jnp.int32)
        u = (u + 0x7FFF + ((u >> 16) & 1)) & (-65536)
        return plsc.bitcast(u, jnp.float32)

    ab_v = [bf16_round(a_buf[pl.ds(k * 16, 16)]) for k in range(16)]

    def chunk(c, _):
        off = c * CH
        cp1 = pltpu.async_copy(x_hbm.at[sidx.at[pl.ds(off, CH)]], xs_b, sem_s)
        cp2 = pltpu.async_copy(x_hbm.at[didx.at[pl.ds(off, CH)]], xd_b, sem_d)
        cp3 = pltpu.async_copy(rn_hbm.at[sidx.at[pl.ds(off, CH)]],
                               rs_b.at[pl.ds(0, CH)], sem_rs)
        cp4 = pltpu.async_copy(rn_hbm.at[didx.at[pl.ds(off, CH)]],
                               rd_b.at[pl.ds(0, CH)], sem_rd)
        cp1.wait()
        cp2.wait()

        cp3.wait()
        cp4.wait()

        def edge(e, _):
            acc = zero_v
            rv = (rs_b[pl.ds(e, 16)] * rd_b[pl.ds(e, 16)])[0]
            for k in range(16):
                xs = xs_b[e, pl.ds(k * 16, 16)]
                xd = xd_b[e, pl.ds(k * 16, 16)]
                h = jnp.maximum(xs * xd, 0.0) * rv
                acc = acc + bf16_round(h) * ab_v[k]
            sm[pl.ds(e * 16, 16)] = acc
            return 0

        lax.fori_loop(0, CH, edge, 0, unroll=False)

        for g in range(CH // 16 + 1):
            gbase = (g * 16 + iota) * 16
            tot = zero_v
            for k in range(16):
                tot = tot + plsc.load_gather(sm, [gbase + k])
            sel = jnp.where(tot > 0, tot, NEG)
            s_buf[pl.ds(off + g * 16, 16)] = sel
        return 0

    lax.fori_loop(0, NCHUNK, chunk, 0, unroll=False)
    pltpu.sync_copy(s_buf.at[pl.ds(0, EPW)], s_hbm.at[pl.ds(base, EPW)])


def _score_sc(x, rn, src, dst, a_vec):
    mesh = plsc.VectorSubcoreMesh(core_axis_name="c", subcore_axis_name="s")
    f = pl.kernel(
        _score_body,
        out_type=jax.ShapeDtypeStruct((E,), jnp.float32),
        mesh=mesh,
        compiler_params=pltpu.CompilerParams(needs_layout_passes=False),
        scratch_types=[
            pltpu.VMEM((EPW,), jnp.int32),
            pltpu.VMEM((EPW,), jnp.int32),
            pltpu.VMEM((CH, D), jnp.float32),
            pltpu.VMEM((CH, D), jnp.float32),
            pltpu.VMEM((CH + 16,), jnp.float32),
            pltpu.VMEM((CH + 16,), jnp.float32),
            pltpu.VMEM((EPW + 16,), jnp.float32),
            pltpu.VMEM(((CH + 16) * 16,), jnp.float32),
            pltpu.VMEM((D,), jnp.float32),
            pltpu.SemaphoreType.DMA,
            pltpu.SemaphoreType.DMA,
            pltpu.SemaphoreType.DMA,
            pltpu.SemaphoreType.DMA,
        ],
    )
    return f(x, rn, src, dst, a_vec)


def _assemble_body(src_hbm, dst_hbm, s_hbm, m_hbm,
                   key_r, dst_r, s_r, lidx, zbuf, bnc0, bnc1, spmem,
                   bsem0, bsem1):
    core = lax.axis_index("c")
    sub = lax.axis_index("s")
    base = sub * EPT
    pltpu.sync_copy(src_hbm.at[pl.ds(base, EPT)], key_r.at[pl.ds(0, EPT)])
    pltpu.sync_copy(dst_hbm.at[pl.ds(base, EPT)], dst_r.at[pl.ds(0, EPT)])
    pltpu.sync_copy(s_hbm.at[pl.ds(base, EPT)], s_r.at[pl.ds(0, EPT)])

    def keyset(i, _):
        key_r[pl.ds(i * 16, 16)] = key_r[pl.ds(i * 16, 16)] * N + dst_r[pl.ds(i * 16, 16)]
        return 0

    lax.fori_loop(0, EPT // 16, keyset, 0, unroll=False)
    neg1 = jnp.full((16,), -1, jnp.int32)
    for i in range(EPT // 16, EPT_PAD // 16):
        key_r[pl.ds(i * 16, 16)] = neg1

    def zset(i, _):
        zbuf[pl.ds(i * 16, 16)] = jnp.zeros((16,), jnp.float32)
        return 0

    lax.fori_loop(0, 5120 // 16, zset, 0, unroll=False)

    zb = sub * ZSPAN

    def zinit(i, _):
        pltpu.sync_copy(zbuf.at[pl.ds(0, 5008)],
                        spmem.at[pl.ds(zb + i * 5008, 5008)])
        return 0

    lax.fori_loop(0, Z_FULL, zinit, 0, unroll=False)
    pltpu.sync_copy(zbuf.at[pl.ds(0, Z_REM)],
                    spmem.at[pl.ds(zb + Z_FULL * 5008, Z_REM)])
    plsc.subcore_barrier()

    iota16 = lax.iota(jnp.int32, 16)

    def block(b, _):
        lo = (b * NC + core) * RB
        lon = lo * N
        hin = jnp.minimum(lon + RB * N, N * N)

        def idxrow(r, _):
            for kk in range(8):
                o = r * 128 + kk * 16
                k = key_r[pl.ds(o, 16)]
                ok = (k >= lon) & (k < hin)
                dmp = DUMP + iota16 + kk * 16
                lidx[r, pl.ds(kk * 16, 16)] = jnp.where(ok, k - lon, dmp)
            return 0

        lax.fori_loop(0, NROW128, idxrow, 0, unroll=False)

        def sadd(j, _):
            pltpu.async_copy(s_r.at[pl.ds(j * 128, 128)],
                             spmem.at[lidx.at[j]], bsem0, add=True)
            return 0

        lax.fori_loop(0, NROW128, sadd, 0, unroll=False)

        def sadd_drain(j, _):
            pltpu.make_async_copy(s_r.at[pl.ds(j * 128, 128)],
                                  spmem.at[lidx.at[j]], bsem0).wait()
            return 0

        lax.fori_loop(0, NROW128, sadd_drain, 0, unroll=False)
        plsc.subcore_barrier()

        sbase = sub * CSPAN
        hbase = lon + sub * CSPAN

        @pl.when(lo < N)
        def _copy_out():
            pend = [None, None]
            for i in range(CSPAN // BNC):
                bb = (bnc0, bnc1)[i % 2]
                if pend[i % 2] is not None:
                    pend[i % 2].wait()
                pltpu.sync_copy(spmem.at[pl.ds(sbase + i * BNC, BNC)], bb)
                pend[i % 2] = pltpu.async_copy(
                    bb, m_hbm.at[pl.ds(hbase + i * BNC, BNC)],
                    (bsem0, bsem1)[i % 2])
            pend[0].wait()
            pend[1].wait()

        plsc.subcore_barrier()

        def szero(j, _):
            pltpu.async_copy(zbuf.at[pl.ds(0, 128)], spmem.at[lidx.at[j]],
                             bsem1)
            return 0

        lax.fori_loop(0, NROW128, szero, 0, unroll=False)

        def szero_drain(j, _):
            pltpu.make_async_copy(zbuf.at[pl.ds(0, 128)],
                                  spmem.at[lidx.at[j]], bsem1).wait()
            return 0

        lax.fori_loop(0, NROW128, szero_drain, 0, unroll=False)
        plsc.subcore_barrier()
        return 0

    lax.fori_loop(0, NBLK_IT, block, 0, unroll=False)


def _assemble_sc(src, dst, s):
    mesh = plsc.VectorSubcoreMesh(core_axis_name="c", subcore_axis_name="s")
    f = pl.kernel(
        _assemble_body,
        out_type=jax.ShapeDtypeStruct((N * N,), jnp.float32),
        mesh=mesh,
        compiler_params=pltpu.CompilerParams(needs_layout_passes=False),
        scratch_types=[
            pltpu.VMEM((EPT_PAD,), jnp.int32),
            pltpu.VMEM((EPT_PAD,), jnp.int32),
            pltpu.VMEM((EPT_PAD,), jnp.float32),
            pltpu.VMEM((NROW128, 128), jnp.int32),
            pltpu.VMEM((5120,), jnp.float32),
            pltpu.VMEM((BNC,), jnp.float32),
            pltpu.VMEM((BNC,), jnp.float32),
            pltpu.VMEM_SHARED((SP_SZ,), jnp.float32),
            pltpu.SemaphoreType.DMA,
            pltpu.SemaphoreType.DMA,
        ],
    )
    return f(src, dst, s)


def _softmax_body(m_ref, o_ref):
    m = m_ref[...]
    mask = m != 0.0
    logits = jnp.where(mask, m, -jnp.inf)
    rowmax = jnp.max(logits, axis=1, keepdims=True)
    safe = jnp.where(jnp.isfinite(rowmax), rowmax, 0.0)
    e = jnp.where(mask, jnp.exp(m - safe), 0.0)
    denom = jnp.sum(e, axis=1, keepdims=True)
    o_ref[...] = jnp.where(denom > 0, e / jnp.where(denom > 0, denom, 1.0), 0.0)


def _row_softmax(M):
    return pl.pallas_call(
        _softmax_body,
        grid=(N // ROWS_SM,),
        in_specs=[pl.BlockSpec((ROWS_SM, N), lambda i: (i, 0))],
        out_specs=pl.BlockSpec((ROWS_SM, N), lambda i: (i, 0)),
        out_shape=jax.ShapeDtypeStruct((N, N), jnp.float32),
    )(M)


def _score_jnp(x, src, dst, a):
    xs = x[src]
    xd = x[dst]
    norm = jnp.sqrt(
        jnp.sum(xs * xs, axis=1, keepdims=True)
        * jnp.sum(xd * xd, axis=1, keepdims=True))
    h = jax.nn.relu(xs * xd / norm)
    s = jnp.squeeze(h @ a)
    return jnp.where(s > 0, s, jnp.full_like(s, NEG))


def kernel(input, edge, W, a):
    x, rn = _project(input, W)
    src = edge[0]
    dst = edge[1]
    s = _score_sc(x, rn.reshape(-1), src, dst, a.reshape(-1))
    M = _assemble_sc(src, dst, s)
    A = _row_softmax(M.reshape(N, N))
    return (x, A)
